# baseline (speedup 1.0000x reference)
import functools

import jax
import jax.numpy as jnp
from jax import lax
from jax.experimental import pallas as pl
from jax.experimental.pallas import tpu as pltpu
from jax.experimental.pallas import tpu_sc as plsc

NC = 2
NS = 16
L = 16
NW = NC * NS

B = 4096
D = 32
BPW = B // NW
DCHUNK = 8


def _complex_score_body(pos_hbm, er_hbm, ei_hbm, rr_hbm, ri_hbm, out_hbm,
                        pos_v, hi_v, ri_v, ti_v,
                        hre_v, him_v, rre_v, rim_v, tre_v, tim_v,
                        out_v, sems):
    wid = lax.axis_index("s") * NC + lax.axis_index("c")
    base = wid * BPW

    pltpu.sync_copy(pos_hbm.at[pl.ds(base * 3, BPW * 3)], pos_v)
    for g in range(BPW // L):
        rows3 = (g * L + lax.iota(jnp.int32, L)) * 3
        for c, dst in ((0, hi_v), (1, ri_v), (2, ti_v)):
            dst[pl.ds(g * L, L)] = plsc.load_gather(pos_v, [rows3 + c])

    pairs = ((er_hbm, hi_v, hre_v, 0), (ei_hbm, hi_v, him_v, 1),
             (rr_hbm, ri_v, rre_v, 2), (ri_hbm, ri_v, rim_v, 3),
             (er_hbm, ti_v, tre_v, 4), (ei_hbm, ti_v, tim_v, 5))

    def chunk_fn(c, carry):
        d0 = c * DCHUNK
        for dd in range(DCHUNK):
            d = d0 + dd
            for tab, idx, dst, s in pairs:
                pltpu.async_copy(tab.at[d].at[idx], dst.at[d], sems[s])
        for dd in range(DCHUNK):
            d = d0 + dd
            for tab, idx, dst, s in pairs:
                pltpu.make_async_copy(tab.at[d].at[idx], dst.at[d],
                                      sems[s]).wait()
        return carry

    lax.fori_loop(0, D // DCHUNK, chunk_fn, 0)

    def group_fn(g, carry):
        s = g * L
        acc = jnp.zeros((L,), jnp.float32)
        for d in range(D):
            hr = hre_v[d, pl.ds(s, L)]
            hi = him_v[d, pl.ds(s, L)]
            rr = rre_v[d, pl.ds(s, L)]
            ri = rim_v[d, pl.ds(s, L)]
            tr = tre_v[d, pl.ds(s, L)]
            ti = tim_v[d, pl.ds(s, L)]
            acc = acc + ((hr * rr - hi * ri) * tr + (hr * ri + hi * rr) * ti)
        out_v[pl.ds(s, L)] = acc
        return carry

    lax.fori_loop(0, BPW // L, group_fn, 0)

    pltpu.sync_copy(out_v, out_hbm.at[pl.ds(base, BPW)])


@jax.jit
def _complex_score(pos_sample, ent_t, ent_im_t, rel_t, rel_im_t):
    mesh = plsc.VectorSubcoreMesh(
        core_axis_name="c", subcore_axis_name="s",
        num_cores=NC, num_subcores=NS)
    run = pl.kernel(
        _complex_score_body,
        out_type=jax.ShapeDtypeStruct((B,), jnp.float32),
        mesh=mesh,
        scratch_types=[
            pltpu.VMEM((BPW * 3,), jnp.int32),
            pltpu.VMEM((BPW,), jnp.int32),
            pltpu.VMEM((BPW,), jnp.int32),
            pltpu.VMEM((BPW,), jnp.int32),
            pltpu.VMEM((D, BPW), jnp.float32),
            pltpu.VMEM((D, BPW), jnp.float32),
            pltpu.VMEM((D, BPW), jnp.float32),
            pltpu.VMEM((D, BPW), jnp.float32),
            pltpu.VMEM((D, BPW), jnp.float32),
            pltpu.VMEM((D, BPW), jnp.float32),
            pltpu.VMEM((BPW,), jnp.float32),
            [pltpu.SemaphoreType.DMA] * 6,
        ],
        compiler_params=pltpu.CompilerParams(
            needs_layout_passes=False, use_tc_tiling_on_sc=False),
    )
    return run(pos_sample.reshape(-1), ent_t, ent_im_t, rel_t, rel_im_t)


def kernel(pos_sample, ent_embd, ent_embd_im, rel_embd, rel_embd_im):
    score = _complex_score(pos_sample, ent_embd.T, ent_embd_im.T,
                           rel_embd.T, rel_embd_im.T)
    return score.reshape(B, 1)


# speedup vs baseline: 6.1487x; 6.1487x over previous
import functools

import jax
import jax.numpy as jnp
from jax import lax
from jax.experimental import pallas as pl
from jax.experimental.pallas import tpu as pltpu
from jax.experimental.pallas import tpu_sc as plsc

NC = 2
NS = 16
L = 16
NW = NC * NS

B = 4096
D = 32
BPW = B // NW


def _complex_score_body(pos_hbm, er_hbm, ei_hbm, rr_hbm, ri_hbm, out_hbm,
                        pos_v, hi_v, ri_v, ti_v,
                        hre_v, him_v, rre_v, rim_v, tre_v, tim_v,
                        half_v, out_v, sems):
    wid = lax.axis_index("s") * NC + lax.axis_index("c")
    base = wid * BPW

    pltpu.sync_copy(pos_hbm.at[pl.ds(base * 3, BPW * 3)], pos_v)
    for g in range(BPW // L):
        rows3 = (g * L + lax.iota(jnp.int32, L)) * 3
        for c, dst in ((0, hi_v), (1, ri_v), (2, ti_v)):
            dst[pl.ds(g * L, L)] = plsc.load_gather(pos_v, [rows3 + c])

    copies = [
        pltpu.async_copy(er_hbm.at[hi_v], hre_v, sems[0]),
        pltpu.async_copy(ei_hbm.at[hi_v], him_v, sems[1]),
        pltpu.async_copy(rr_hbm.at[ri_v], rre_v, sems[2]),
        pltpu.async_copy(ri_hbm.at[ri_v], rim_v, sems[3]),
        pltpu.async_copy(er_hbm.at[ti_v], tre_v, sems[4]),
        pltpu.async_copy(ei_hbm.at[ti_v], tim_v, sems[5]),
    ]
    for cp in copies:
        cp.wait()

    def row_fn(i, carry):
        hr0 = hre_v[i, pl.ds(0, L)]
        hr1 = hre_v[i, pl.ds(L, L)]
        hi0 = him_v[i, pl.ds(0, L)]
        hi1 = him_v[i, pl.ds(L, L)]
        rr0 = rre_v[i, pl.ds(0, L)]
        rr1 = rre_v[i, pl.ds(L, L)]
        ri0 = rim_v[i, pl.ds(0, L)]
        ri1 = rim_v[i, pl.ds(L, L)]
        tr0 = tre_v[i, pl.ds(0, L)]
        tr1 = tre_v[i, pl.ds(L, L)]
        ti0 = tim_v[i, pl.ds(0, L)]
        ti1 = tim_v[i, pl.ds(L, L)]
        s0 = (hr0 * rr0 - hi0 * ri0) * tr0 + (hr0 * ri0 + hi0 * rr0) * ti0
        s1 = (hr1 * rr1 - hi1 * ri1) * tr1 + (hr1 * ri1 + hi1 * rr1) * ti1
        half_v[pl.ds(i * L, L)] = s0 + s1
        return carry

    lax.fori_loop(0, BPW, row_fn, 0)

    for g in range(BPW // L):
        rows16 = (g * L + lax.iota(jnp.int32, L)) * L
        acc = plsc.load_gather(half_v, [rows16])
        for j in range(1, L):
            acc = acc + plsc.load_gather(half_v, [rows16 + j])
        out_v[pl.ds(g * L, L)] = acc

    pltpu.sync_copy(out_v, out_hbm.at[pl.ds(base, BPW)])


@jax.jit
def _complex_score(pos_sample, ent_embd, ent_embd_im, rel_embd, rel_embd_im):
    mesh = plsc.VectorSubcoreMesh(
        core_axis_name="c", subcore_axis_name="s",
        num_cores=NC, num_subcores=NS)
    run = pl.kernel(
        _complex_score_body,
        out_type=jax.ShapeDtypeStruct((B,), jnp.float32),
        mesh=mesh,
        scratch_types=[
            pltpu.VMEM((BPW * 3,), jnp.int32),
            pltpu.VMEM((BPW,), jnp.int32),
            pltpu.VMEM((BPW,), jnp.int32),
            pltpu.VMEM((BPW,), jnp.int32),
            pltpu.VMEM((BPW, D), jnp.float32),
            pltpu.VMEM((BPW, D), jnp.float32),
            pltpu.VMEM((BPW, D), jnp.float32),
            pltpu.VMEM((BPW, D), jnp.float32),
            pltpu.VMEM((BPW, D), jnp.float32),
            pltpu.VMEM((BPW, D), jnp.float32),
            pltpu.VMEM((BPW * L,), jnp.float32),
            pltpu.VMEM((BPW,), jnp.float32),
            [pltpu.SemaphoreType.DMA] * 6,
        ],
        compiler_params=pltpu.CompilerParams(
            needs_layout_passes=False, use_tc_tiling_on_sc=False),
    )
    return run(pos_sample.reshape(-1), ent_embd, ent_embd_im,
               rel_embd, rel_embd_im)


def kernel(pos_sample, ent_embd, ent_embd_im, rel_embd, rel_embd_im):
    score = _complex_score(pos_sample, ent_embd, ent_embd_im,
                           rel_embd, rel_embd_im)
    return score.reshape(B, 1)
